# Initial kernel scaffold; baseline (speedup 1.0000x reference)
import jax
import jax.numpy as jnp
from jax.experimental import pallas as pl


def kernel(x, edge_index, W1, b1, W2, b2, Wfc, bfc):
    raise NotImplementedError("write your pallas kernel here")


# speedup vs baseline: 42.3302x; 42.3302x over previous
import functools

import jax
import jax.numpy as jnp
from jax import lax
from jax.experimental import pallas as pl
from jax.experimental.pallas import tpu as pltpu
from jax.experimental.pallas import tpu_sc as plsc

N = 10000
D = 128
H = 16
NPAD = 10240
NC = 2
NS = 16
NW = NC * NS
EB = 128
GP = 79
ET = GP * EB
EPAD = NW * ET
NPT = NPAD // NS
BR = 512


def _mm1_body(x_ref, w_ref, o_ref):
    o_ref[...] = jnp.dot(x_ref[...], w_ref[...],
                         preferred_element_type=jnp.float32)


def _mm1(xpad, W1):
    return pl.pallas_call(
        _mm1_body,
        grid=(NPAD // BR,),
        in_specs=[pl.BlockSpec((BR, D), lambda i: (i, 0)),
                  pl.BlockSpec((D, H), lambda i: (0, 0))],
        out_specs=pl.BlockSpec((BR, H), lambda i: (i, 0)),
        out_shape=jax.ShapeDtypeStruct((NPAD, H), jnp.float32),
    )(xpad, W1)


def _comb1_body(p0, p1, y1, iv, b1, w2, o):
    h = jnp.maximum((p0[...] + p1[...] + y1[...]) * iv[...] + b1[...], 0.0)
    o[...] = jnp.dot(h, w2[...], preferred_element_type=jnp.float32) * iv[...]


def _comb1(p0, p1, y1, iv, b1, W2):
    row = pl.BlockSpec((BR, H), lambda i: (i, 0))
    return pl.pallas_call(
        _comb1_body,
        grid=(NPAD // BR,),
        in_specs=[row, row, row,
                  pl.BlockSpec((BR, 1), lambda i: (i, 0)),
                  pl.BlockSpec((1, H), lambda i: (0, 0)),
                  pl.BlockSpec((H, H), lambda i: (0, 0))],
        out_specs=row,
        out_shape=jax.ShapeDtypeStruct((NPAD, H), jnp.float32),
    )(p0, p1, y1, iv, b1, W2)


def _comb2_body(p0, p1, y2, iv, b2, wfc, bfc, o):
    h = jnp.maximum((p0[...] + p1[...] + y2[...]) * iv[...] + b2[...], 0.0)
    z = jnp.dot(h, wfc[...], preferred_element_type=jnp.float32) + bfc[...]
    m = jnp.max(z, axis=1, keepdims=True)
    lse = m + jnp.log(jnp.sum(jnp.exp(z - m), axis=1, keepdims=True))
    o[...] = z - lse


def _comb2(p0, p1, y2, iv, b2, Wfc, bfc):
    no = Wfc.shape[1]
    row = pl.BlockSpec((BR, H), lambda i: (i, 0))
    return pl.pallas_call(
        _comb2_body,
        grid=(NPAD // BR,),
        in_specs=[row, row, row,
                  pl.BlockSpec((BR, 1), lambda i: (i, 0)),
                  pl.BlockSpec((1, H), lambda i: (0, 0)),
                  pl.BlockSpec((H, no), lambda i: (0, 0)),
                  pl.BlockSpec((1, no), lambda i: (0, 0))],
        out_specs=pl.BlockSpec((BR, no), lambda i: (i, 0)),
        out_shape=jax.ShapeDtypeStruct((NPAD, no), jnp.float32),
    )(p0, p1, y2, iv, b2, Wfc, bfc)


def _rsqrt16(d):
    bits = lax.bitcast_convert_type(d, jnp.int32)
    bits = jnp.int32(0x5F3759DF) - (bits >> 1)
    r = lax.bitcast_convert_type(bits, jnp.float32)
    for _ in range(3):
        r = r * (1.5 - 0.5 * d * r * r)
    return r


def _sc1_body(xw, srcr, dstr, aggp, inv_o, y1_o,
              deg_sh, y_sh, agg_sh, idx_a, idx_b, xw_t, deg_t, inv_t, y_t,
              ones_t, zero_t, rows_t):
    c = lax.axis_index("c")
    s = lax.axis_index("s")
    wid = s * NC + c
    nbase = s * NPT

    def _init_zero(i, carry):
        zero_t[i] = jnp.zeros((H,), jnp.float32)
        return carry
    lax.fori_loop(0, EB, _init_zero, 0)
    for i in range(EB // 16):
        ones_t[pl.ds(i * 16, 16)] = jnp.ones((16,), jnp.float32)

    pltpu.sync_copy(xw.at[pl.ds(nbase, NPT)], xw_t)
    pltpu.sync_copy(dstr.at[s * 2], idx_a)
    pltpu.sync_copy(dstr.at[s * 2 + 1], idx_b)

    for j in range(NPT // EB):
        pltpu.sync_copy(ones_t, deg_sh.at[pl.ds(nbase + j * EB, EB)])
        pltpu.sync_copy(zero_t, agg_sh.at[pl.ds(nbase + j * EB, EB)])

    plsc.subcore_barrier()

    for g in range(GP):
        pltpu.sync_copy(ones_t, deg_sh.at[idx_a.at[g]], add=True)
        pltpu.sync_copy(ones_t, deg_sh.at[idx_b.at[g]], add=True)

    plsc.subcore_barrier()

    pltpu.sync_copy(deg_sh.at[pl.ds(nbase, NPT)], deg_t)

    def _newton(i, carry):
        dd = deg_t[pl.ds(i * 16, 16)]
        inv_t[pl.ds(i * 16, 16)] = _rsqrt16(dd)
        return carry
    lax.fori_loop(0, NPT // 16, _newton, 0)

    def _scale(i, carry):
        viv = inv_t[pl.ds(i * 16, 16)]
        base = i * 16
        for j in range(16):
            y_t[base + j] = xw_t[base + j] * viv[j]
        return carry
    lax.fori_loop(0, NPT // 16, _scale, 0)

    pltpu.sync_copy(y_t, y_sh.at[pl.ds(nbase, NPT)])

    @pl.when(c == 0)
    def _():
        pltpu.sync_copy(inv_t, inv_o.at[pl.ds(nbase, NPT)])
        pltpu.sync_copy(y_t, y1_o.at[pl.ds(nbase, NPT)])

    pltpu.sync_copy(srcr.at[wid], idx_a)
    pltpu.sync_copy(dstr.at[wid], idx_b)

    plsc.subcore_barrier()

    for g in range(GP):
        pltpu.sync_copy(y_sh.at[idx_a.at[g]], rows_t)
        pltpu.sync_copy(rows_t, agg_sh.at[idx_b.at[g]], add=True)

    plsc.subcore_barrier()

    pltpu.sync_copy(agg_sh.at[pl.ds(nbase, NPT)],
                    aggp.at[pl.ds(c * NPAD + nbase, NPT)])


def _sc_layer1(xw, srcr, dstr):
    mesh = plsc.VectorSubcoreMesh(core_axis_name="c", subcore_axis_name="s",
                                  num_cores=NC, num_subcores=NS)
    kfn = pl.kernel(
        _sc1_body,
        out_type=(
            jax.ShapeDtypeStruct((NC * NPAD, H), jnp.float32),
            jax.ShapeDtypeStruct((NPAD,), jnp.float32),
            jax.ShapeDtypeStruct((NPAD, H), jnp.float32),
        ),
        mesh=mesh,
        scratch_types=[
            pltpu.VMEM_SHARED((NPAD,), jnp.float32),
            pltpu.VMEM_SHARED((NPAD, H), jnp.float32),
            pltpu.VMEM_SHARED((NPAD, H), jnp.float32),
            pltpu.VMEM((GP, EB), jnp.int32),
            pltpu.VMEM((GP, EB), jnp.int32),
            pltpu.VMEM((NPT, H), jnp.float32),
            pltpu.VMEM((NPT,), jnp.float32),
            pltpu.VMEM((NPT,), jnp.float32),
            pltpu.VMEM((NPT, H), jnp.float32),
            pltpu.VMEM((EB,), jnp.float32),
            pltpu.VMEM((EB, H), jnp.float32),
            pltpu.VMEM((EB, H), jnp.float32),
        ],
        compiler_params=pltpu.CompilerParams(use_tc_tiling_on_sc=False),
    )
    return kfn(xw, srcr, dstr)


def _sc2_body(y2, srcr, dstr, aggp,
              y_sh, agg_sh, idx_a, idx_b, y_t, zero_t, rows_t):
    c = lax.axis_index("c")
    s = lax.axis_index("s")
    wid = s * NC + c
    nbase = s * NPT

    def _init_zero(i, carry):
        zero_t[i] = jnp.zeros((H,), jnp.float32)
        return carry
    lax.fori_loop(0, EB, _init_zero, 0)

    pltpu.sync_copy(y2.at[pl.ds(nbase, NPT)], y_t)
    pltpu.sync_copy(y_t, y_sh.at[pl.ds(nbase, NPT)])
    for j in range(NPT // EB):
        pltpu.sync_copy(zero_t, agg_sh.at[pl.ds(nbase + j * EB, EB)])
    pltpu.sync_copy(srcr.at[wid], idx_a)
    pltpu.sync_copy(dstr.at[wid], idx_b)

    plsc.subcore_barrier()

    for g in range(GP):
        pltpu.sync_copy(y_sh.at[idx_a.at[g]], rows_t)
        pltpu.sync_copy(rows_t, agg_sh.at[idx_b.at[g]], add=True)

    plsc.subcore_barrier()

    pltpu.sync_copy(agg_sh.at[pl.ds(nbase, NPT)],
                    aggp.at[pl.ds(c * NPAD + nbase, NPT)])


def _sc_layer2(y2, srcr, dstr):
    mesh = plsc.VectorSubcoreMesh(core_axis_name="c", subcore_axis_name="s",
                                  num_cores=NC, num_subcores=NS)
    kfn = pl.kernel(
        _sc2_body,
        out_type=jax.ShapeDtypeStruct((NC * NPAD, H), jnp.float32),
        mesh=mesh,
        scratch_types=[
            pltpu.VMEM_SHARED((NPAD, H), jnp.float32),
            pltpu.VMEM_SHARED((NPAD, H), jnp.float32),
            pltpu.VMEM((GP, EB), jnp.int32),
            pltpu.VMEM((GP, EB), jnp.int32),
            pltpu.VMEM((NPT, H), jnp.float32),
            pltpu.VMEM((EB, H), jnp.float32),
            pltpu.VMEM((EB, H), jnp.float32),
        ],
        compiler_params=pltpu.CompilerParams(use_tc_tiling_on_sc=False),
    )
    return kfn(y2, srcr, dstr)


def kernel(x, edge_index, W1, b1, W2, b2, Wfc, bfc):
    n = x.shape[0]
    e = edge_index.shape[1]
    src = edge_index[0]
    dst = edge_index[1]

    xpad = jnp.pad(x, ((0, NPAD - n), (0, 0)))
    extra = (jnp.arange(EPAD - e, dtype=jnp.int32) % (NPAD - n)) + n
    srcp = jnp.concatenate([src, extra]).reshape(NW, GP, EB)
    dstp = jnp.concatenate([dst, extra]).reshape(NW, GP, EB)

    xw1 = _mm1(xpad, W1)
    aggp1, inv, y1 = _sc_layer1(xw1, srcp, dstp)
    iv = inv.reshape(NPAD, 1)
    y2 = _comb1(aggp1[:NPAD], aggp1[NPAD:], y1, iv, b1.reshape(1, H), W2)
    aggp2 = _sc_layer2(y2, srcp, dstp)
    out = _comb2(aggp2[:NPAD], aggp2[NPAD:], y2, iv, b2.reshape(1, H),
                 Wfc, bfc.reshape(1, Wfc.shape[1]))
    return out[:n]


# speedup vs baseline: 48.7477x; 1.1516x over previous
import functools

import jax
import jax.numpy as jnp
from jax import lax
from jax.experimental import pallas as pl
from jax.experimental.pallas import tpu as pltpu
from jax.experimental.pallas import tpu_sc as plsc

N = 10000
D = 128
H = 16
NPAD = 10240
NC = 2
NS = 16
NW = NC * NS
EB = 128
GP = 79
ET = GP * EB
EPAD = NW * ET
NPT = NPAD // NS
BR = 512


def _mm1_body(x_ref, w_ref, o_ref):
    o_ref[...] = jnp.dot(x_ref[...], w_ref[...],
                         preferred_element_type=jnp.float32)


def _mm1(xpad, W1):
    return pl.pallas_call(
        _mm1_body,
        grid=(NPAD // BR,),
        in_specs=[pl.BlockSpec((BR, D), lambda i: (i, 0)),
                  pl.BlockSpec((D, H), lambda i: (0, 0))],
        out_specs=pl.BlockSpec((BR, H), lambda i: (i, 0)),
        out_shape=jax.ShapeDtypeStruct((NPAD, H), jnp.float32),
    )(xpad, W1)


def _comb1_body(p0, p1, y1, iv, b1, w2, o):
    h = jnp.maximum((p0[...] + p1[...] + y1[...]) * iv[...] + b1[...], 0.0)
    o[...] = jnp.dot(h, w2[...], preferred_element_type=jnp.float32) * iv[...]


def _comb1(p0, p1, y1, iv, b1, W2):
    row = pl.BlockSpec((BR, H), lambda i: (i, 0))
    return pl.pallas_call(
        _comb1_body,
        grid=(NPAD // BR,),
        in_specs=[row, row, row,
                  pl.BlockSpec((BR, 1), lambda i: (i, 0)),
                  pl.BlockSpec((1, H), lambda i: (0, 0)),
                  pl.BlockSpec((H, H), lambda i: (0, 0))],
        out_specs=row,
        out_shape=jax.ShapeDtypeStruct((NPAD, H), jnp.float32),
    )(p0, p1, y1, iv, b1, W2)


def _comb2_body(p0, p1, y2, iv, b2, wfc, bfc, o):
    h = jnp.maximum((p0[...] + p1[...] + y2[...]) * iv[...] + b2[...], 0.0)
    z = jnp.dot(h, wfc[...], preferred_element_type=jnp.float32) + bfc[...]
    m = jnp.max(z, axis=1, keepdims=True)
    lse = m + jnp.log(jnp.sum(jnp.exp(z - m), axis=1, keepdims=True))
    o[...] = z - lse


def _comb2(p0, p1, y2, iv, b2, Wfc, bfc):
    no = Wfc.shape[1]
    row = pl.BlockSpec((BR, H), lambda i: (i, 0))
    return pl.pallas_call(
        _comb2_body,
        grid=(NPAD // BR,),
        in_specs=[row, row, row,
                  pl.BlockSpec((BR, 1), lambda i: (i, 0)),
                  pl.BlockSpec((1, H), lambda i: (0, 0)),
                  pl.BlockSpec((H, no), lambda i: (0, 0)),
                  pl.BlockSpec((1, no), lambda i: (0, 0))],
        out_specs=pl.BlockSpec((BR, no), lambda i: (i, 0)),
        out_shape=jax.ShapeDtypeStruct((NPAD, no), jnp.float32),
    )(p0, p1, y2, iv, b2, Wfc, bfc)


NBUF = 4


def _edge_pass(y_sh, agg_sh, src_t, dst_t, rows, gsems, ssems):
    gd, sd = {}, {}
    for g in range(GP + 3):
        if g < GP:
            b = g % NBUF
            if g >= NBUF:
                sd[g - NBUF].wait()
            gd[g] = pltpu.async_copy(y_sh.at[src_t.at[g]], rows[b], gsems[b])
        gp = g - 3
        if gp >= 0:
            b = gp % NBUF
            gd[gp].wait()
            sd[gp] = pltpu.async_copy(rows[b], agg_sh.at[dst_t.at[gp]],
                                      ssems[b], add=True)
    for gp in range(GP - NBUF, GP):
        sd[gp].wait()


def _deg_pass(deg_sh, dst_refs, ones_t, dsems):
    d = {}
    i = 0
    for g in range(GP):
        for ref in dst_refs:
            b = i % NBUF
            if i >= NBUF:
                d[i - NBUF].wait()
            d[i] = pltpu.async_copy(ones_t, deg_sh.at[ref.at[g]], dsems[b],
                                    add=True)
            i += 1
    for j in range(max(0, i - NBUF), i):
        d[j].wait()


def _rsqrt16(d):
    bits = lax.bitcast_convert_type(d, jnp.int32)
    bits = jnp.int32(0x5F3759DF) - (bits >> 1)
    r = lax.bitcast_convert_type(bits, jnp.float32)
    for _ in range(3):
        r = r * (1.5 - 0.5 * d * r * r)
    return r


def _sc1_body(xw, srcr, dstr, aggp, inv_o, y1_o,
              deg_sh, y_sh, agg_sh, idx_a, idx_b, xw_t, deg_t, inv_t, y_t,
              ones_t, zero_t, r0, r1, r2, r3,
              g0, g1, g2, g3, s0, s1, s2, s3):
    rows = (r0, r1, r2, r3)
    gsems = (g0, g1, g2, g3)
    ssems = (s0, s1, s2, s3)
    c = lax.axis_index("c")
    s = lax.axis_index("s")
    wid = s * NC + c
    nbase = s * NPT

    def _init_zero(i, carry):
        zero_t[i] = jnp.zeros((H,), jnp.float32)
        return carry
    lax.fori_loop(0, EB, _init_zero, 0)
    for i in range(EB // 16):
        ones_t[pl.ds(i * 16, 16)] = jnp.ones((16,), jnp.float32)

    pltpu.sync_copy(xw.at[pl.ds(nbase, NPT)], xw_t)
    pltpu.sync_copy(dstr.at[s * 2], idx_a)
    pltpu.sync_copy(dstr.at[s * 2 + 1], idx_b)

    for j in range(NPT // EB):
        pltpu.sync_copy(ones_t, deg_sh.at[pl.ds(nbase + j * EB, EB)])
        pltpu.sync_copy(zero_t, agg_sh.at[pl.ds(nbase + j * EB, EB)])

    plsc.subcore_barrier()

    _deg_pass(deg_sh, (idx_a, idx_b), ones_t, gsems)

    plsc.subcore_barrier()

    pltpu.sync_copy(deg_sh.at[pl.ds(nbase, NPT)], deg_t)

    def _newton(i, carry):
        dd = deg_t[pl.ds(i * 16, 16)]
        inv_t[pl.ds(i * 16, 16)] = _rsqrt16(dd)
        return carry
    lax.fori_loop(0, NPT // 16, _newton, 0)

    def _scale(i, carry):
        viv = inv_t[pl.ds(i * 16, 16)]
        base = i * 16
        for j in range(16):
            y_t[base + j] = xw_t[base + j] * viv[j]
        return carry
    lax.fori_loop(0, NPT // 16, _scale, 0)

    pltpu.sync_copy(y_t, y_sh.at[pl.ds(nbase, NPT)])

    @pl.when(c == 0)
    def _():
        pltpu.sync_copy(inv_t, inv_o.at[pl.ds(nbase, NPT)])
        pltpu.sync_copy(y_t, y1_o.at[pl.ds(nbase, NPT)])

    pltpu.sync_copy(srcr.at[wid], idx_a)
    pltpu.sync_copy(dstr.at[wid], idx_b)

    plsc.subcore_barrier()

    _edge_pass(y_sh, agg_sh, idx_a, idx_b, rows, gsems, ssems)

    plsc.subcore_barrier()

    pltpu.sync_copy(agg_sh.at[pl.ds(nbase, NPT)],
                    aggp.at[pl.ds(c * NPAD + nbase, NPT)])


def _sc_layer1(xw, srcr, dstr):
    mesh = plsc.VectorSubcoreMesh(core_axis_name="c", subcore_axis_name="s",
                                  num_cores=NC, num_subcores=NS)
    kfn = pl.kernel(
        _sc1_body,
        out_type=(
            jax.ShapeDtypeStruct((NC * NPAD, H), jnp.float32),
            jax.ShapeDtypeStruct((NPAD,), jnp.float32),
            jax.ShapeDtypeStruct((NPAD, H), jnp.float32),
        ),
        mesh=mesh,
        scratch_types=[
            pltpu.VMEM_SHARED((NPAD,), jnp.float32),
            pltpu.VMEM_SHARED((NPAD, H), jnp.float32),
            pltpu.VMEM_SHARED((NPAD, H), jnp.float32),
            pltpu.VMEM((GP, EB), jnp.int32),
            pltpu.VMEM((GP, EB), jnp.int32),
            pltpu.VMEM((NPT, H), jnp.float32),
            pltpu.VMEM((NPT,), jnp.float32),
            pltpu.VMEM((NPT,), jnp.float32),
            pltpu.VMEM((NPT, H), jnp.float32),
            pltpu.VMEM((EB,), jnp.float32),
            pltpu.VMEM((EB, H), jnp.float32),
            pltpu.VMEM((EB, H), jnp.float32),
            pltpu.VMEM((EB, H), jnp.float32),
            pltpu.VMEM((EB, H), jnp.float32),
            pltpu.VMEM((EB, H), jnp.float32),
            pltpu.SemaphoreType.DMA,
            pltpu.SemaphoreType.DMA,
            pltpu.SemaphoreType.DMA,
            pltpu.SemaphoreType.DMA,
            pltpu.SemaphoreType.DMA,
            pltpu.SemaphoreType.DMA,
            pltpu.SemaphoreType.DMA,
            pltpu.SemaphoreType.DMA,
        ],
        compiler_params=pltpu.CompilerParams(use_tc_tiling_on_sc=False),
    )
    return kfn(xw, srcr, dstr)


def _sc2_body(y2, srcr, dstr, aggp,
              y_sh, agg_sh, idx_a, idx_b, y_t, zero_t, r0, r1, r2, r3,
              g0, g1, g2, g3, s0, s1, s2, s3):
    rows = (r0, r1, r2, r3)
    gsems = (g0, g1, g2, g3)
    ssems = (s0, s1, s2, s3)
    c = lax.axis_index("c")
    s = lax.axis_index("s")
    wid = s * NC + c
    nbase = s * NPT

    def _init_zero(i, carry):
        zero_t[i] = jnp.zeros((H,), jnp.float32)
        return carry
    lax.fori_loop(0, EB, _init_zero, 0)

    pltpu.sync_copy(y2.at[pl.ds(nbase, NPT)], y_t)
    pltpu.sync_copy(y_t, y_sh.at[pl.ds(nbase, NPT)])
    for j in range(NPT // EB):
        pltpu.sync_copy(zero_t, agg_sh.at[pl.ds(nbase + j * EB, EB)])
    pltpu.sync_copy(srcr.at[wid], idx_a)
    pltpu.sync_copy(dstr.at[wid], idx_b)

    plsc.subcore_barrier()

    _edge_pass(y_sh, agg_sh, idx_a, idx_b, rows, gsems, ssems)

    plsc.subcore_barrier()

    pltpu.sync_copy(agg_sh.at[pl.ds(nbase, NPT)],
                    aggp.at[pl.ds(c * NPAD + nbase, NPT)])


def _sc_layer2(y2, srcr, dstr):
    mesh = plsc.VectorSubcoreMesh(core_axis_name="c", subcore_axis_name="s",
                                  num_cores=NC, num_subcores=NS)
    kfn = pl.kernel(
        _sc2_body,
        out_type=jax.ShapeDtypeStruct((NC * NPAD, H), jnp.float32),
        mesh=mesh,
        scratch_types=[
            pltpu.VMEM_SHARED((NPAD, H), jnp.float32),
            pltpu.VMEM_SHARED((NPAD, H), jnp.float32),
            pltpu.VMEM((GP, EB), jnp.int32),
            pltpu.VMEM((GP, EB), jnp.int32),
            pltpu.VMEM((NPT, H), jnp.float32),
            pltpu.VMEM((EB, H), jnp.float32),
            pltpu.VMEM((EB, H), jnp.float32),
            pltpu.VMEM((EB, H), jnp.float32),
            pltpu.VMEM((EB, H), jnp.float32),
            pltpu.VMEM((EB, H), jnp.float32),
            pltpu.SemaphoreType.DMA,
            pltpu.SemaphoreType.DMA,
            pltpu.SemaphoreType.DMA,
            pltpu.SemaphoreType.DMA,
            pltpu.SemaphoreType.DMA,
            pltpu.SemaphoreType.DMA,
            pltpu.SemaphoreType.DMA,
            pltpu.SemaphoreType.DMA,
        ],
        compiler_params=pltpu.CompilerParams(use_tc_tiling_on_sc=False),
    )
    return kfn(y2, srcr, dstr)


def kernel(x, edge_index, W1, b1, W2, b2, Wfc, bfc):
    n = x.shape[0]
    e = edge_index.shape[1]
    src = edge_index[0]
    dst = edge_index[1]

    xpad = jnp.pad(x, ((0, NPAD - n), (0, 0)))
    extra = (jnp.arange(EPAD - e, dtype=jnp.int32) % (NPAD - n)) + n
    srcp = jnp.concatenate([src, extra]).reshape(NW, GP, EB)
    dstp = jnp.concatenate([dst, extra]).reshape(NW, GP, EB)

    xw1 = _mm1(xpad, W1)
    aggp1, inv, y1 = _sc_layer1(xw1, srcp, dstp)
    iv = inv.reshape(NPAD, 1)
    y2 = _comb1(aggp1[:NPAD], aggp1[NPAD:], y1, iv, b1.reshape(1, H), W2)
    aggp2 = _sc_layer2(y2, srcp, dstp)
    out = _comb2(aggp2[:NPAD], aggp2[NPAD:], y2, iv, b2.reshape(1, H),
                 Wfc, bfc.reshape(1, Wfc.shape[1]))
    return out[:n]


# speedup vs baseline: 70.7586x; 1.4515x over previous
import functools

import jax
import jax.numpy as jnp
from jax import lax
from jax.experimental import pallas as pl
from jax.experimental.pallas import tpu as pltpu
from jax.experimental.pallas import tpu_sc as plsc

N = 10000
D = 128
H = 16
NPAD = 10240
NC = 2
NS = 16
NW = NC * NS
EB = 128
GP = 79
ET = GP * EB
EPAD = NW * ET
NPT = NPAD // NS
BR = 512


NPP = NPAD // 8
BP = 256


def _mm1_body(x_ref, w_ref, o_ref):
    o_ref[...] = jnp.dot(x_ref[...], w_ref[...],
                         preferred_element_type=jnp.float32)


def _mm1(xpad, W1p):
    return pl.pallas_call(
        _mm1_body,
        grid=(NPAD // BR,),
        in_specs=[pl.BlockSpec((BR, D), lambda i: (i, 0)),
                  pl.BlockSpec((D, 128), lambda i: (0, 0))],
        out_specs=pl.BlockSpec((BR, 128), lambda i: (i, 0)),
        out_shape=jax.ShapeDtypeStruct((NPAD, 128), jnp.float32),
    )(xpad, W1p)


def _comb1_body(p0, p1, y1, iv, b1, w2, o):
    h = jnp.maximum((p0[...] + p1[...] + y1[...]) * iv[...] + b1[...], 0.0)
    o[...] = jnp.dot(h, w2[...], preferred_element_type=jnp.float32) * iv[...]


def _comb1(aggp, y1, iv, b1t, w2bd):
    row = pl.BlockSpec((BP, 128), lambda i: (i, 0))
    return pl.pallas_call(
        _comb1_body,
        grid=(NPP // BP,),
        in_specs=[row,
                  pl.BlockSpec((BP, 128), lambda i: (i + NPP // BP, 0)),
                  row, row,
                  pl.BlockSpec((1, 128), lambda i: (0, 0)),
                  pl.BlockSpec((128, 128), lambda i: (0, 0))],
        out_specs=row,
        out_shape=jax.ShapeDtypeStruct((NPP, 128), jnp.float32),
    )(aggp, aggp, y1, iv, b1t, w2bd)


def _comb2_body(p0, p1, y2, iv, b2, wfcbd, bfct, sw, o):
    h = jnp.maximum((p0[...] + p1[...] + y2[...]) * iv[...] + b2[...], 0.0)
    z = jnp.dot(h, wfcbd[...], preferred_element_type=jnp.float32) + bfct[...]
    zs = jnp.dot(z, sw[...], preferred_element_type=jnp.float32)
    m = jnp.maximum(z, zs)
    o[...] = z - (m + jnp.log(jnp.exp(z - m) + jnp.exp(zs - m)))


def _comb2(aggp, y2, iv, b2t, wfcbd, bfct, sw):
    no16 = wfcbd.shape[1]
    row = pl.BlockSpec((BP, 128), lambda i: (i, 0))
    return pl.pallas_call(
        _comb2_body,
        grid=(NPP // BP,),
        in_specs=[row,
                  pl.BlockSpec((BP, 128), lambda i: (i + NPP // BP, 0)),
                  row, row,
                  pl.BlockSpec((1, 128), lambda i: (0, 0)),
                  pl.BlockSpec((128, no16), lambda i: (0, 0)),
                  pl.BlockSpec((1, no16), lambda i: (0, 0)),
                  pl.BlockSpec((no16, no16), lambda i: (0, 0))],
        out_specs=pl.BlockSpec((BP, no16), lambda i: (i, 0)),
        out_shape=jax.ShapeDtypeStruct((NPP, no16), jnp.float32),
    )(aggp, aggp, y2, iv, b2t, wfcbd, bfct, sw)


NBUF = 4


def _edge_pass(y_sh, agg_sh, src_t, dst_t, rows, gsems, ssems):
    gd, sd = {}, {}
    for g in range(GP + 3):
        if g < GP:
            b = g % NBUF
            if g >= NBUF:
                sd[g - NBUF].wait()
            gd[g] = pltpu.async_copy(y_sh.at[src_t.at[g]], rows[b], gsems[b])
        gp = g - 3
        if gp >= 0:
            b = gp % NBUF
            gd[gp].wait()
            sd[gp] = pltpu.async_copy(rows[b], agg_sh.at[dst_t.at[gp]],
                                      ssems[b], add=True)
    for gp in range(GP - NBUF, GP):
        sd[gp].wait()


def _deg_pass(deg_sh, dst_refs, ones_t, dsems):
    d = {}
    i = 0
    for g in range(GP):
        for ref in dst_refs:
            b = i % NBUF
            if i >= NBUF:
                d[i - NBUF].wait()
            d[i] = pltpu.async_copy(ones_t, deg_sh.at[ref.at[g]], dsems[b],
                                    add=True)
            i += 1
    for j in range(max(0, i - NBUF), i):
        d[j].wait()


def _rsqrt16(d):
    bits = lax.bitcast_convert_type(d, jnp.int32)
    bits = jnp.int32(0x5F3759DF) - (bits >> 1)
    r = lax.bitcast_convert_type(bits, jnp.float32)
    for _ in range(3):
        r = r * (1.5 - 0.5 * d * r * r)
    return r


def _sc1_body(xw, srcr, dstr, aggp, invb_o, y1_o,
              deg_sh, agg_sh, idx_a, idx_b, xw_t, deg_t, inv_t,
              invb_t, y_t, ones_t, zero_t, r0, r1, r2, r3,
              g0, g1, g2, g3, s0, s1, s2, s3):
    rows = (r0, r1, r2, r3)
    gsems = (g0, g1, g2, g3)
    ssems = (s0, s1, s2, s3)
    c = lax.axis_index("c")
    s = lax.axis_index("s")
    wid = s * NC + c
    nbase = s * NPT

    def _init_zero(i, carry):
        zero_t[i] = jnp.zeros((H,), jnp.float32)
        return carry
    lax.fori_loop(0, EB, _init_zero, 0)
    for i in range(EB // 16):
        ones_t[pl.ds(i * 16, 16)] = jnp.ones((16,), jnp.float32)

    pltpu.sync_copy(xw.at[pl.ds(nbase, NPT), pl.ds(0, H)], xw_t)
    pltpu.sync_copy(dstr.at[s * 2], idx_a)
    pltpu.sync_copy(dstr.at[s * 2 + 1], idx_b)

    for j in range(NPT // EB):
        pltpu.sync_copy(ones_t, deg_sh.at[pl.ds(nbase + j * EB, EB)])
        pltpu.sync_copy(zero_t, agg_sh.at[pl.ds(nbase + j * EB, EB)])

    plsc.subcore_barrier()

    _deg_pass(deg_sh, (idx_a, idx_b), ones_t, gsems)

    plsc.subcore_barrier()

    pltpu.sync_copy(deg_sh.at[pl.ds(nbase, NPT)], deg_t)

    def _newton(i, carry):
        dd = deg_t[pl.ds(i * 16, 16)]
        inv_t[pl.ds(i * 16, 16)] = _rsqrt16(dd)
        return carry
    lax.fori_loop(0, NPT // 16, _newton, 0)

    def _scale(i, carry):
        viv = inv_t[pl.ds(i * 16, 16)]
        base = i * 16
        for j in range(16):
            s = viv[j]
            y_t[base + j] = xw_t[base + j] * s
            invb_t[base + j] = jnp.full((16,), s, jnp.float32)
        return carry
    lax.fori_loop(0, NPT // 16, _scale, 0)

    pltpu.sync_copy(y_t, y1_o.at[pl.ds(nbase, NPT)])

    @pl.when(c == 0)
    def _():
        pltpu.sync_copy(invb_t, invb_o.at[pl.ds(nbase, NPT)])

    pltpu.sync_copy(srcr.at[wid], idx_a)
    pltpu.sync_copy(dstr.at[wid], idx_b)

    plsc.subcore_barrier()

    _edge_pass(y1_o, agg_sh, idx_a, idx_b, rows, gsems, ssems)

    plsc.subcore_barrier()

    pltpu.sync_copy(agg_sh.at[pl.ds(nbase, NPT)],
                    aggp.at[pl.ds(c * NPAD + nbase, NPT)])


def _sc_layer1(xw, srcr, dstr):
    mesh = plsc.VectorSubcoreMesh(core_axis_name="c", subcore_axis_name="s",
                                  num_cores=NC, num_subcores=NS)
    kfn = pl.kernel(
        _sc1_body,
        out_type=(
            jax.ShapeDtypeStruct((NC * NPAD, H), jnp.float32),
            jax.ShapeDtypeStruct((NPAD, H), jnp.float32),
            jax.ShapeDtypeStruct((NPAD, H), jnp.float32),
        ),
        mesh=mesh,
        scratch_types=[
            pltpu.VMEM_SHARED((NPAD,), jnp.float32),
            pltpu.VMEM_SHARED((NPAD, H), jnp.float32),
            pltpu.VMEM((GP, EB), jnp.int32),
            pltpu.VMEM((GP, EB), jnp.int32),
            pltpu.VMEM((NPT, H), jnp.float32),
            pltpu.VMEM((NPT,), jnp.float32),
            pltpu.VMEM((NPT,), jnp.float32),
            pltpu.VMEM((NPT, H), jnp.float32),
            pltpu.VMEM((NPT, H), jnp.float32),
            pltpu.VMEM((EB,), jnp.float32),
            pltpu.VMEM((EB, H), jnp.float32),
            pltpu.VMEM((EB, H), jnp.float32),
            pltpu.VMEM((EB, H), jnp.float32),
            pltpu.VMEM((EB, H), jnp.float32),
            pltpu.VMEM((EB, H), jnp.float32),
            pltpu.SemaphoreType.DMA,
            pltpu.SemaphoreType.DMA,
            pltpu.SemaphoreType.DMA,
            pltpu.SemaphoreType.DMA,
            pltpu.SemaphoreType.DMA,
            pltpu.SemaphoreType.DMA,
            pltpu.SemaphoreType.DMA,
            pltpu.SemaphoreType.DMA,
        ],
        compiler_params=pltpu.CompilerParams(use_tc_tiling_on_sc=False),
    )
    return kfn(xw, srcr, dstr)


def _sc2_body(y2, srcr, dstr, aggp,
              agg_sh, idx_a, idx_b, zero_t, r0, r1, r2, r3,
              g0, g1, g2, g3, s0, s1, s2, s3):
    rows = (r0, r1, r2, r3)
    gsems = (g0, g1, g2, g3)
    ssems = (s0, s1, s2, s3)
    c = lax.axis_index("c")
    s = lax.axis_index("s")
    wid = s * NC + c
    nbase = s * NPT

    def _init_zero(i, carry):
        zero_t[i] = jnp.zeros((H,), jnp.float32)
        return carry
    lax.fori_loop(0, EB, _init_zero, 0)

    for j in range(NPT // EB):
        pltpu.sync_copy(zero_t, agg_sh.at[pl.ds(nbase + j * EB, EB)])
    pltpu.sync_copy(srcr.at[wid], idx_a)
    pltpu.sync_copy(dstr.at[wid], idx_b)

    plsc.subcore_barrier()

    _edge_pass(y2, agg_sh, idx_a, idx_b, rows, gsems, ssems)

    plsc.subcore_barrier()

    pltpu.sync_copy(agg_sh.at[pl.ds(nbase, NPT)],
                    aggp.at[pl.ds(c * NPAD + nbase, NPT)])


def _sc_layer2(y2, srcr, dstr):
    mesh = plsc.VectorSubcoreMesh(core_axis_name="c", subcore_axis_name="s",
                                  num_cores=NC, num_subcores=NS)
    kfn = pl.kernel(
        _sc2_body,
        out_type=jax.ShapeDtypeStruct((NC * NPAD, H), jnp.float32),
        mesh=mesh,
        scratch_types=[
            pltpu.VMEM_SHARED((NPAD, H), jnp.float32),
            pltpu.VMEM((GP, EB), jnp.int32),
            pltpu.VMEM((GP, EB), jnp.int32),
            pltpu.VMEM((EB, H), jnp.float32),
            pltpu.VMEM((EB, H), jnp.float32),
            pltpu.VMEM((EB, H), jnp.float32),
            pltpu.VMEM((EB, H), jnp.float32),
            pltpu.VMEM((EB, H), jnp.float32),
            pltpu.SemaphoreType.DMA,
            pltpu.SemaphoreType.DMA,
            pltpu.SemaphoreType.DMA,
            pltpu.SemaphoreType.DMA,
            pltpu.SemaphoreType.DMA,
            pltpu.SemaphoreType.DMA,
            pltpu.SemaphoreType.DMA,
            pltpu.SemaphoreType.DMA,
        ],
        compiler_params=pltpu.CompilerParams(use_tc_tiling_on_sc=False),
    )
    return kfn(y2, srcr, dstr)


def kernel(x, edge_index, W1, b1, W2, b2, Wfc, bfc):
    n = x.shape[0]
    e = edge_index.shape[1]
    src = edge_index[0]
    dst = edge_index[1]

    xpad = jnp.pad(x, ((0, NPAD - n), (0, 0)))
    extra = (jnp.arange(EPAD - e, dtype=jnp.int32) % (NPAD - n)) + n
    srcp = jnp.concatenate([src, extra]).reshape(NW, GP, EB)
    dstp = jnp.concatenate([dst, extra]).reshape(NW, GP, EB)

    no = Wfc.shape[1]
    eye8 = jnp.eye(8, dtype=jnp.float32)

    xw128 = _mm1(xpad, jnp.pad(W1, ((0, 0), (0, 128 - H))))
    aggp1, invb, y1 = _sc_layer1(xw128, srcp, dstp)
    ivp = invb.reshape(NPP, 128)
    y2p = _comb1(aggp1.reshape(2 * NPP, 128), y1.reshape(NPP, 128), ivp,
                 jnp.tile(b1, 8).reshape(1, 128), jnp.kron(eye8, W2))
    aggp2 = _sc_layer2(y2p.reshape(NPAD, H), srcp, dstp)
    outp = _comb2(aggp2.reshape(2 * NPP, 128), y2p, ivp,
                  jnp.tile(b2, 8).reshape(1, 128), jnp.kron(eye8, Wfc),
                  jnp.tile(bfc, 8).reshape(1, 8 * no),
                  jnp.kron(eye8, jnp.array([[0.0, 1.0], [1.0, 0.0]],
                                           jnp.float32)))
    return outp.reshape(NPAD, no)[:n]
